# baseline (speedup 1.0000x reference)
import functools
import math

import jax
import jax.numpy as jnp
from jax import lax
from jax.experimental import pallas as pl
from jax.experimental.pallas import tpu as pltpu
from jax.experimental.pallas import tpu_sc as plsc

_DIM = 64
_SCALE = math.sqrt(_DIM)

_NC = 2
_NS = 16
_NW = _NC * _NS
_NB = 4


@functools.cache
def _build(NBATCH, T):
  assert NBATCH % _NW == 0
  rows_per_w = NBATCH // _NW
  b_per_w = rows_per_w * T
  n_chunks = rows_per_w
  assert n_chunks % _NB == 0 and n_chunks // _NB >= 2
  widths = []
  t = T
  while t > 0:
    w = min(128, t)
    widths.append(w)
    t -= w

  mesh = plsc.VectorSubcoreMesh(core_axis_name="c", subcore_axis_name="s")

  def body(x_hbm, table_hbm, out_hbm, idx_v, bufs, gsems, wsems):
    wid = lax.axis_index("s") * _NC + lax.axis_index("c")
    base = pl.multiple_of(wid * b_per_w, b_per_w)
    brow0 = pl.multiple_of(wid * rows_per_w, rows_per_w)

    pltpu.sync_copy(x_hbm.at[pl.ds(base, b_per_w)], idx_v)

    def gcopies(g, nb):
      out = []
      o = 0
      for w in widths:
        out.append(pltpu.make_async_copy(
            table_hbm.at[idx_v.at[pl.ds(pl.multiple_of(g * T + o, 8), w)]],
            bufs[nb].at[0, pl.ds(o, w)],
            gsems[nb],
        ))
        o += w
      return out

    def wcopy(g, nb):
      return pltpu.make_async_copy(
          bufs[nb],
          out_hbm.at[pl.ds(brow0 + g, 1)],
          wsems[nb],
      )

    def scale(nb):
      def srow(r, c):
        for rr in range(2):
          for u in range(_DIM // 16):
            sl = pl.ds(u * 16, 16)
            bufs[nb][0, r * 2 + rr, sl] = bufs[nb][0, r * 2 + rr, sl] * _SCALE
        return c

      lax.fori_loop(0, T // 2, srow, 0)

    def chunk(g, nb, fire_ahead, wait_prev=True):
      for cp in gcopies(g, nb):
        cp.wait()
      scale(nb)
      wcopy(g, nb).start()
      if fire_ahead:
        nb2 = (nb + _NB - 1) % _NB
        if wait_prev:
          wcopy(g - 1, nb2).wait()
        for cp in gcopies(g + _NB - 1, nb2):
          cp.start()

    for nb in range(_NB - 1):
      for cp in gcopies(nb, nb):
        cp.start()

    chunk(0, 0, True, wait_prev=False)
    for nb in range(1, _NB):
      chunk(nb, nb, True)

    def outer(i, carry):
      g0 = i * _NB
      for nb in range(_NB):
        chunk(g0 + nb, nb, True)
      return carry

    lax.fori_loop(1, n_chunks // _NB - 1, outer, 0)

    g0 = n_chunks - _NB
    chunk(g0, 0, True)
    for nb in range(1, _NB):
      chunk(g0 + nb, nb, False)
    for nb in range(_NB):
      wcopy(g0 + nb, nb).wait()

  return pl.kernel(
      body,
      out_type=jax.ShapeDtypeStruct((NBATCH, T, _DIM), jnp.float32),
      mesh=mesh,
      compiler_params=pltpu.CompilerParams(use_tc_tiling_on_sc=False),
      scratch_types=[
          pltpu.VMEM((b_per_w,), jnp.int32),
          [pltpu.VMEM((1, T, _DIM), jnp.float32) for _ in range(_NB)],
          [pltpu.SemaphoreType.DMA for _ in range(_NB)],
          [pltpu.SemaphoreType.DMA for _ in range(_NB)],
      ],
  )


def kernel(x, table):
  NBATCH, T = x.shape
  x1 = x.reshape(-1).astype(jnp.int32)
  return _build(NBATCH, T)(x1, table.astype(jnp.float32))


# speedup vs baseline: 1.0024x; 1.0024x over previous
import functools
import math

import jax
import jax.numpy as jnp
from jax import lax
from jax.experimental import pallas as pl
from jax.experimental.pallas import tpu as pltpu
from jax.experimental.pallas import tpu_sc as plsc

_DIM = 64
_SCALE = math.sqrt(_DIM)

_NC = 2
_NS = 16
_NW = _NC * _NS
_NB = 4
_C = 128


@functools.cache
def _build(n_lookups):
  per_w = n_lookups // _NW
  n_chunks = per_w // _C
  assert n_lookups % _NW == 0 and per_w % _C == 0
  assert n_chunks % _NB == 0 and n_chunks // _NB >= 2

  mesh = plsc.VectorSubcoreMesh(core_axis_name="c", subcore_axis_name="s")

  def body(x_hbm, table_hbm, out_hbm, idx_v, gbufs, obufs, gsems, wsems):
    wid = lax.axis_index("s") * _NC + lax.axis_index("c")
    base = wid * per_w

    pltpu.sync_copy(x_hbm.at[pl.ds(base, per_w)], idx_v)

    def gcopy(g, nb):
      return pltpu.make_async_copy(
          table_hbm.at[idx_v.at[pl.ds(g * _C, _C)]], gbufs[nb], gsems[nb])

    def wcopy(g, nb):
      return pltpu.make_async_copy(
          obufs[nb], out_hbm.at[pl.ds(base + g * _C, _C)], wsems[nb])

    def compute(nb):
      def rows(r, c):
        for rr in range(4):
          i = r * 4 + rr
          for q in range(4):
            sl = pl.ds(q * 16, 16)
            obufs[nb][i, sl] = gbufs[nb][i, sl] * _SCALE
        return c
      lax.fori_loop(0, _C // 4, rows, 0)

    def chunk(g, nb, fire, wait_prev=True):
      gcopy(g, nb).wait()
      if fire:
        gcopy(g + _NB - 1, (nb + _NB - 1) % _NB).start()
      if wait_prev:
        wcopy(g - _NB, nb).wait()
      compute(nb)
      wcopy(g, nb).start()

    for nb in range(_NB - 1):
      gcopy(nb, nb).start()

    for nb in range(_NB):
      chunk(nb, nb, True, wait_prev=False)

    def outer(i, carry):
      g0 = i * _NB
      for nb in range(_NB):
        chunk(g0 + nb, nb, True)
      return carry

    lax.fori_loop(1, n_chunks // _NB - 1, outer, 0)

    g0 = n_chunks - _NB
    chunk(g0, 0, True)
    for nb in range(1, _NB):
      chunk(g0 + nb, nb, False)
    for nb in range(_NB):
      wcopy(g0 + nb, nb).wait()

  return pl.kernel(
      body,
      out_type=jax.ShapeDtypeStruct((n_lookups, _DIM), jnp.float32),
      mesh=mesh,
      compiler_params=pltpu.CompilerParams(use_tc_tiling_on_sc=False),
      scratch_types=[
          pltpu.VMEM((per_w,), jnp.int32),
          [pltpu.VMEM((_C, _DIM), jnp.float32) for _ in range(_NB)],
          [pltpu.VMEM((_C, _DIM), jnp.float32) for _ in range(_NB)],
          [pltpu.SemaphoreType.DMA for _ in range(_NB)],
          [pltpu.SemaphoreType.DMA for _ in range(_NB)],
      ],
  )


def kernel(x, table):
  nb, t = x.shape
  xf = x.astype(jnp.int32).reshape(nb * t)
  p = _build(nb * t)(xf, table.astype(jnp.float32))
  return p.reshape(nb, t, _DIM)
